# baseline (speedup 1.0000x reference)
import functools

import jax
import jax.numpy as jnp
from jax import lax
from jax.experimental import pallas as pl
from jax.experimental.pallas import tpu as pltpu
from jax.experimental.pallas import tpu_sc as plsc

B = 4096 * 200
D = 64
NC = 2
NS = 16
NW = NC * NS
BPW = B // NW
K = 128
NBUF = 4
NSET = BPW // (K * NBUF)

_mesh = plsc.VectorSubcoreMesh(core_axis_name="c", subcore_axis_name="s")

NROW = 1000000
RPAD = 1000064
TBLK = RPAD // 128
TPW = (TBLK + NW - 1) // NW


@functools.partial(
    pl.kernel,
    mesh=_mesh,
    out_type=jax.ShapeDtypeStruct((RPAD * D,), jnp.float32),
    scratch_types=[
        pltpu.VMEM((8, 8, 128), jnp.float32),
        pltpu.VMEM((128 * D,), jnp.float32),
        pltpu.SemaphoreType.DMA,
    ],
    compiler_params=pltpu.CompilerParams(
        use_tc_tiling_on_sc=True, disable_bounds_checks=True,
        needs_layout_passes=False),
)
def _detile(wT, out, tiles_v, blk_v, sem):
    wid = lax.axis_index("s") * NC + lax.axis_index("c")
    lane = lax.iota(jnp.int32, 16)
    bases = [(lane + 16 * j) * D for j in range(8)]

    def block(j, carry):
        t = wid * TPW + j

        @pl.when(t < TBLK)
        def _():
            for g in range(8):
                pltpu.make_async_copy(
                    wT.at[pl.ds(g * 8, 8), pl.ds(t * 128, 128)],
                    tiles_v.at[g], sem).start()
            for g in range(8):
                pltpu.make_async_copy(
                    wT.at[pl.ds(g * 8, 8), pl.ds(t * 128, 128)],
                    tiles_v.at[g], sem).wait()
            for g in range(8):
                for k in range(8):
                    c = 8 * g + k
                    for j16 in range(8):
                        v = tiles_v[g, k, pl.ds(16 * j16, 16)]
                        plsc.store_scatter(blk_v, [bases[j16] + c], v)
            pltpu.sync_copy(blk_v, out.at[pl.ds(t * (128 * D), 128 * D)])
        return carry

    lax.fori_loop(0, TPW, block, None)


@functools.partial(
    pl.kernel,
    mesh=_mesh,
    out_type=jax.ShapeDtypeStruct((B, D), jnp.float32),
    scratch_types=[
        pltpu.VMEM((BPW,), jnp.int32),
        pltpu.VMEM((2, NBUF, K, D), jnp.float32),
        pltpu.SemaphoreType.DMA,
        pltpu.SemaphoreType.DMA,
    ],
    compiler_params=pltpu.CompilerParams(use_tc_tiling_on_sc=False),
)
def _emb_gather(idx_hbm, table_hbm, out_hbm, idx_v, rows_v, gsem, osem):
    wid = lax.axis_index("s") * NC + lax.axis_index("c")
    base = wid * BPW
    pltpu.sync_copy(idx_hbm.at[pl.ds(base, BPW)], idx_v)

    def fire_gathers(s, half):
        for b in range(NBUF):
            pltpu.make_async_copy(
                table_hbm.at[idx_v.at[pl.ds(s * (K * NBUF) + b * K, K)]],
                rows_v.at[half, b], gsem).start()

    def drain_gathers(half):
        for b in range(NBUF):
            pltpu.make_async_copy(
                table_hbm.at[idx_v.at[pl.ds(b * K, K)]],
                rows_v.at[half, b], gsem).wait()

    def fire_writes(s, half):
        for b in range(NBUF):
            pltpu.make_async_copy(
                rows_v.at[half, b],
                out_hbm.at[pl.ds(base + s * (K * NBUF) + b * K, K)],
                osem).start()

    def drain_writes(half):
        for b in range(NBUF):
            pltpu.make_async_copy(
                rows_v.at[half, b],
                out_hbm.at[pl.ds(base, K)], osem).wait()

    fire_gathers(0, 0)

    def step2(i, carry):
        g = i * 2
        for p in range(2):
            s = g + p
            cur, other = p, 1 - p

            @pl.when(s > 0)
            def _():
                drain_writes(other)

            @pl.when(s + 1 < NSET)
            def _():
                fire_gathers(s + 1, other)

            drain_gathers(cur)
            fire_writes(s, cur)
        return carry

    lax.fori_loop(0, NSET // 2, step2, None)
    drain_writes(1)


def kernel(idxs, weight):
    flat = idxs.reshape(-1).astype(jnp.int32)
    tab = _detile(weight.T).reshape(RPAD, D)
    out = _emb_gather(flat, tab)
    return out.reshape(idxs.shape + (weight.shape[-1],))


# speedup vs baseline: 1.4693x; 1.4693x over previous
import functools

import jax
import jax.numpy as jnp
from jax import lax
from jax.experimental import pallas as pl
from jax.experimental.pallas import tpu as pltpu
from jax.experimental.pallas import tpu_sc as plsc

B = 4096 * 200
D = 64
NC = 2
NS = 16
NW = NC * NS
BPW = B // NW
K = 128
NBUF = 4
NSET = BPW // (K * NBUF)

_mesh = plsc.VectorSubcoreMesh(core_axis_name="c", subcore_axis_name="s")

NROW = 1000000
RPAD = 1000064
TBLK = RPAD // 128
TPW = (TBLK + NW - 1) // NW


@functools.partial(
    pl.kernel,
    mesh=_mesh,
    out_type=jax.ShapeDtypeStruct((RPAD * D,), jnp.float32),
    scratch_types=[
        pltpu.VMEM((64, 128), jnp.float32),
        pltpu.VMEM((64, 128), jnp.float32),
        pltpu.VMEM((128 * D,), jnp.float32),
        pltpu.VMEM((128 * D,), jnp.float32),
        pltpu.SemaphoreType.DMA,
        pltpu.SemaphoreType.DMA,
    ],
    compiler_params=pltpu.CompilerParams(
        use_tc_tiling_on_sc=True, disable_bounds_checks=True,
        needs_layout_passes=False),
)
def _detile(wT, out, tiles_a, tiles_b, blk_a, blk_b, isem, osem):
    tiles_v = [tiles_a, tiles_b]
    blk_v = [blk_a, blk_b]
    wid = lax.axis_index("s") * NC + lax.axis_index("c")
    t0 = wid * TPW
    hi = jnp.minimum(t0 + TPW, TBLK)
    lane = lax.iota(jnp.int32, 16)
    bases = [(lane + 16 * j) * D for j in range(8)]

    def in_copy(t, p):
        return pltpu.make_async_copy(
            wT.at[:, pl.ds(t * 128, 128)], tiles_v[p], isem)

    def out_copy(t, p):
        return pltpu.make_async_copy(
            blk_v[p], out.at[pl.ds(t * (128 * D), 128 * D)], osem)

    @pl.when(t0 < hi)
    def _():
        in_copy(t0, 0).start()

    def step(j, carry):
        for p in range(2):
            t = t0 + j * 2 + p

            @pl.when(t < hi)
            def _():
                in_copy(t, p).wait()

                @pl.when(t + 1 < hi)
                def _():
                    in_copy(t + 1, 1 - p).start()

                @pl.when(j * 2 + p >= 2)
                def _():
                    out_copy(t, p).wait()

                @plsc.parallel_loop(0, 64, 1, unroll=4)
                def _(i):
                    for j16 in range(8):
                        v = tiles_v[p][i, pl.ds(16 * j16, 16)]
                        plsc.store_scatter(blk_v[p], [bases[j16] + i], v)

                out_copy(t, p).start()
        return carry

    lax.fori_loop(0, (TPW + 1) // 2, step, None)
    for p in range(2):
        out_copy(t0, p).wait()


@functools.partial(
    pl.kernel,
    mesh=_mesh,
    out_type=jax.ShapeDtypeStruct((B, D), jnp.float32),
    scratch_types=[
        pltpu.VMEM((BPW,), jnp.int32),
        pltpu.VMEM((2, NBUF, K, D), jnp.float32),
        pltpu.SemaphoreType.DMA,
        pltpu.SemaphoreType.DMA,
    ],
    compiler_params=pltpu.CompilerParams(use_tc_tiling_on_sc=False),
)
def _emb_gather(idx_hbm, table_hbm, out_hbm, idx_v, rows_v, gsem, osem):
    wid = lax.axis_index("s") * NC + lax.axis_index("c")
    base = wid * BPW
    pltpu.sync_copy(idx_hbm.at[pl.ds(base, BPW)], idx_v)

    def fire_gathers(s, half):
        for b in range(NBUF):
            pltpu.make_async_copy(
                table_hbm.at[idx_v.at[pl.ds(s * (K * NBUF) + b * K, K)]],
                rows_v.at[half, b], gsem).start()

    def drain_gathers(half):
        for b in range(NBUF):
            pltpu.make_async_copy(
                table_hbm.at[idx_v.at[pl.ds(b * K, K)]],
                rows_v.at[half, b], gsem).wait()

    def fire_writes(s, half):
        for b in range(NBUF):
            pltpu.make_async_copy(
                rows_v.at[half, b],
                out_hbm.at[pl.ds(base + s * (K * NBUF) + b * K, K)],
                osem).start()

    def drain_writes(half):
        for b in range(NBUF):
            pltpu.make_async_copy(
                rows_v.at[half, b],
                out_hbm.at[pl.ds(base, K)], osem).wait()

    fire_gathers(0, 0)

    def step2(i, carry):
        g = i * 2
        for p in range(2):
            s = g + p
            cur, other = p, 1 - p

            @pl.when(s > 0)
            def _():
                drain_writes(other)

            @pl.when(s + 1 < NSET)
            def _():
                fire_gathers(s + 1, other)

            drain_gathers(cur)
            fire_writes(s, cur)
        return carry

    lax.fori_loop(0, NSET // 2, step2, None)
    drain_writes(1)


def kernel(idxs, weight):
    flat = idxs.reshape(-1).astype(jnp.int32)
    tab = _detile(weight.T).reshape(RPAD, D)
    out = _emb_gather(flat, tab)
    return out.reshape(idxs.shape + (weight.shape[-1],))


# speedup vs baseline: 1.4698x; 1.0003x over previous
import functools

import jax
import jax.numpy as jnp
from jax import lax
from jax.experimental import pallas as pl
from jax.experimental.pallas import tpu as pltpu
from jax.experimental.pallas import tpu_sc as plsc

B = 4096 * 200
D = 64
NC = 2
NS = 16
NW = NC * NS
BPW = B // NW
K = 128
NBUF = 4
NSET = BPW // (K * NBUF)

_mesh = plsc.VectorSubcoreMesh(core_axis_name="c", subcore_axis_name="s")

NROW = 1000000
RPAD = 1000064
TBLK = RPAD // 128
TPW = (TBLK + NW - 1) // NW


@functools.partial(
    pl.kernel,
    mesh=_mesh,
    out_type=jax.ShapeDtypeStruct((RPAD * D,), jnp.float32),
    scratch_types=[
        pltpu.VMEM((64, 128), jnp.float32),
        pltpu.VMEM((64, 128), jnp.float32),
        pltpu.VMEM((128 * D,), jnp.float32),
        pltpu.VMEM((128 * D,), jnp.float32),
        pltpu.SemaphoreType.DMA,
        pltpu.SemaphoreType.DMA,
    ],
    compiler_params=pltpu.CompilerParams(
        use_tc_tiling_on_sc=True, disable_bounds_checks=True,
        needs_layout_passes=False),
)
def _detile(wT, out, tiles_a, tiles_b, blk_a, blk_b, isem, osem):
    tiles_v = [tiles_a, tiles_b]
    blk_v = [blk_a, blk_b]
    wid = lax.axis_index("s") * NC + lax.axis_index("c")
    t0 = wid * TPW
    hi = jnp.minimum(t0 + TPW, TBLK)
    lane = lax.iota(jnp.int32, 16)
    bases = [(lane + 16 * j) * D for j in range(8)]

    def in_copy(t, p):
        return pltpu.make_async_copy(
            wT.at[:, pl.ds(t * 128, 128)], tiles_v[p], isem)

    def out_copy(t, p):
        return pltpu.make_async_copy(
            blk_v[p], out.at[pl.ds(t * (128 * D), 128 * D)], osem)

    @pl.when(t0 < hi)
    def _():
        in_copy(t0, 0).start()

    def step(j, carry):
        for p in range(2):
            t = t0 + j * 2 + p

            @pl.when(t < hi)
            def _():
                in_copy(t, p).wait()

                @pl.when(t + 1 < hi)
                def _():
                    in_copy(t + 1, 1 - p).start()

                @pl.when(j * 2 + p >= 2)
                def _():
                    out_copy(t, p).wait()

                @plsc.parallel_loop(0, 64, 1, unroll=8)
                def _(i):
                    for j16 in range(8):
                        v = tiles_v[p][i, pl.ds(16 * j16, 16)]
                        plsc.store_scatter(blk_v[p], [bases[j16] + i], v)

                out_copy(t, p).start()
        return carry

    lax.fori_loop(0, (TPW + 1) // 2, step, None)
    for p in range(2):
        out_copy(t0, p).wait()


@functools.partial(
    pl.kernel,
    mesh=_mesh,
    out_type=jax.ShapeDtypeStruct((B, D), jnp.float32),
    scratch_types=[
        pltpu.VMEM((BPW,), jnp.int32),
        pltpu.VMEM((2, NBUF, K, D), jnp.float32),
        pltpu.SemaphoreType.DMA,
        pltpu.SemaphoreType.DMA,
    ],
    compiler_params=pltpu.CompilerParams(use_tc_tiling_on_sc=False),
)
def _emb_gather(idx_hbm, table_hbm, out_hbm, idx_v, rows_v, gsem, osem):
    wid = lax.axis_index("s") * NC + lax.axis_index("c")
    base = wid * BPW
    pltpu.sync_copy(idx_hbm.at[pl.ds(base, BPW)], idx_v)

    def fire_gathers(s, half):
        for b in range(NBUF):
            pltpu.make_async_copy(
                table_hbm.at[idx_v.at[pl.ds(s * (K * NBUF) + b * K, K)]],
                rows_v.at[half, b], gsem).start()

    def drain_gathers(half):
        for b in range(NBUF):
            pltpu.make_async_copy(
                table_hbm.at[idx_v.at[pl.ds(b * K, K)]],
                rows_v.at[half, b], gsem).wait()

    def fire_writes(s, half):
        for b in range(NBUF):
            pltpu.make_async_copy(
                rows_v.at[half, b],
                out_hbm.at[pl.ds(base + s * (K * NBUF) + b * K, K)],
                osem).start()

    def drain_writes(half):
        for b in range(NBUF):
            pltpu.make_async_copy(
                rows_v.at[half, b],
                out_hbm.at[pl.ds(base, K)], osem).wait()

    fire_gathers(0, 0)

    def step2(i, carry):
        g = i * 2
        for p in range(2):
            s = g + p
            cur, other = p, 1 - p

            @pl.when(s > 0)
            def _():
                drain_writes(other)

            @pl.when(s + 1 < NSET)
            def _():
                fire_gathers(s + 1, other)

            drain_gathers(cur)
            fire_writes(s, cur)
        return carry

    lax.fori_loop(0, NSET // 2, step2, None)
    drain_writes(1)


def kernel(idxs, weight):
    flat = idxs.reshape(-1).astype(jnp.int32)
    tab = _detile(weight.T).reshape(RPAD, D)
    out = _emb_gather(flat, tab)
    return out.reshape(idxs.shape + (weight.shape[-1],))


# speedup vs baseline: 1.7777x; 1.2095x over previous
import functools

import jax
import jax.numpy as jnp
from jax import lax
from jax.experimental import pallas as pl
from jax.experimental.pallas import tpu as pltpu
from jax.experimental.pallas import tpu_sc as plsc

B = 4096 * 200
D = 64
NC = 2
NS = 16
NW = NC * NS
BPW = B // NW
K = 128
NBUF = 4
NSET = BPW // (K * NBUF)

_mesh = plsc.VectorSubcoreMesh(core_axis_name="c", subcore_axis_name="s")


@functools.partial(
    pl.kernel,
    mesh=_mesh,
    out_type=jax.ShapeDtypeStruct((B, D), jnp.float32),
    scratch_types=[
        pltpu.VMEM((BPW,), jnp.int32),
        pltpu.VMEM((2, NBUF, K, D), jnp.float32),
        pltpu.SemaphoreType.DMA,
        pltpu.SemaphoreType.DMA,
    ],
    compiler_params=pltpu.CompilerParams(use_tc_tiling_on_sc=False),
)
def _emb_gather(idx_hbm, table_hbm, out_hbm, idx_v, rows_v, gsem, osem):
    wid = lax.axis_index("s") * NC + lax.axis_index("c")
    base = wid * BPW
    pltpu.sync_copy(idx_hbm.at[pl.ds(base, BPW)], idx_v)

    def fire_gathers(s, half):
        for b in range(NBUF):
            pltpu.make_async_copy(
                table_hbm.at[idx_v.at[pl.ds(s * (K * NBUF) + b * K, K)]],
                rows_v.at[half, b], gsem).start()

    def drain_gathers(half):
        for b in range(NBUF):
            pltpu.make_async_copy(
                table_hbm.at[idx_v.at[pl.ds(b * K, K)]],
                rows_v.at[half, b], gsem).wait()

    def fire_writes(s, half):
        for b in range(NBUF):
            pltpu.make_async_copy(
                rows_v.at[half, b],
                out_hbm.at[pl.ds(base + s * (K * NBUF) + b * K, K)],
                osem).start()

    def drain_writes(half):
        for b in range(NBUF):
            pltpu.make_async_copy(
                rows_v.at[half, b],
                out_hbm.at[pl.ds(base, K)], osem).wait()

    fire_gathers(0, 0)

    def step2(i, carry):
        g = i * 2
        for p in range(2):
            s = g + p
            cur, other = p, 1 - p

            @pl.when(s > 0)
            def _():
                drain_writes(other)

            @pl.when(s + 1 < NSET)
            def _():
                fire_gathers(s + 1, other)

            drain_gathers(cur)
            fire_writes(s, cur)
        return carry

    lax.fori_loop(0, NSET // 2, step2, None)
    drain_writes(1)


def kernel(idxs, weight):
    flat = idxs.reshape(-1).astype(jnp.int32)
    out = _emb_gather(flat, weight)
    return out.reshape(idxs.shape + (weight.shape[-1],))
